# baseline (device time: 68616 ns/iter reference)
import jax
import jax.numpy as jnp
from jax import lax
from jax.experimental import pallas as pl
from jax.experimental.pallas import tpu as pltpu

N_DEV = 4
SQ = 256
SKV_SHARD = 4096
HQ = 8
DH = 128
DM = HQ * DH
CW = DM + DH
SCALE = 0.08838834764831843
HALF = SQ // 2
KG = SKV_SHARD // 4


def kernel(x, Wq, K_ext, V_ext, Wo):
    def body(x_ref, wq_ref, k_ref, v_ref, wo_ref, out_ref,
             comm, ctx_ref, qs_ref, ss, rs):
        h = pl.program_id(0)
        my_pos = lax.axis_index("i")
        left = lax.rem(my_pos + N_DEV - 1, N_DEV)
        right = lax.rem(my_pos + 1, N_DEV)

        @pl.when(h == 0)
        def _():
            barrier_sem = pltpu.get_barrier_semaphore()
            for nbr in (left, right):
                pl.semaphore_signal(
                    barrier_sem, inc=1,
                    device_id=(nbr,), device_id_type=pl.DeviceIdType.MESH,
                )
            pl.semaphore_wait(barrier_sem, 2)
            qs_ref[:, :] = jnp.dot(
                x_ref[0], wq_ref[:, :],
                preferred_element_type=jnp.float32) * SCALE

        for j in range(4):
            qhj = qs_ref[j * 64:(j + 1) * 64, pl.ds(h * DH, DH)]
            khj = k_ref[0, :, j, :, :].reshape(KG, DH)
            vhj = v_ref[0, :, j, :, :].reshape(KG, DH)
            s = lax.dot_general(
                qhj, khj, (((1,), (1,)), ((), ())),
                preferred_element_type=jnp.float32)
            w = jnp.exp(s)
            l = jnp.sum(w, axis=1, keepdims=True)
            o = jnp.dot(w, vhj, preferred_element_type=jnp.float32)
            rows = pl.ds(j * 64, 64)
            comm[0, rows, pl.ds(h * DH, DH)] = o.astype(jnp.bfloat16)
            lane = lax.broadcasted_iota(jnp.int32, (64, HQ), 1)
            lrow = comm[0, rows, DM:DM + HQ]
            comm[0, rows, DM:DM + HQ] = jnp.where(
                lane == h, jnp.broadcast_to(l, (64, HQ)), lrow
            ).astype(jnp.bfloat16)

        @pl.when(h == HQ - 1)
        def _():
            r0 = pltpu.make_async_remote_copy(
                src_ref=comm.at[0], dst_ref=comm.at[1],
                send_sem=ss.at[0], recv_sem=rs.at[0],
                device_id=(right,), device_id_type=pl.DeviceIdType.MESH,
            )
            l0 = pltpu.make_async_remote_copy(
                src_ref=comm.at[0], dst_ref=comm.at[2],
                send_sem=ss.at[1], recv_sem=rs.at[1],
                device_id=(left,), device_id_type=pl.DeviceIdType.MESH,
            )
            r0.start()
            l0.start()

            r0.wait_recv()
            r1 = pltpu.make_async_remote_copy(
                src_ref=comm.at[1, pl.ds(0, HALF)],
                dst_ref=comm.at[3, pl.ds(0, HALF)],
                send_sem=ss.at[2], recv_sem=rs.at[2],
                device_id=(right,), device_id_type=pl.DeviceIdType.MESH,
            )
            r1.start()

            l0.wait_recv()
            l1 = pltpu.make_async_remote_copy(
                src_ref=comm.at[2, pl.ds(HALF, HALF)],
                dst_ref=comm.at[3, pl.ds(HALF, HALF)],
                send_sem=ss.at[3], recv_sem=rs.at[3],
                device_id=(left,), device_id_type=pl.DeviceIdType.MESH,
            )
            l1.start()

            part = (comm[0, :, :].astype(jnp.float32)
                    + comm[1, :, :].astype(jnp.float32)
                    + comm[2, :, :].astype(jnp.float32))

            r1.wait_recv()
            l1.wait_recv()
            tot = part + comm[3, :, :].astype(jnp.float32)

            for hh in range(HQ):
                ctx_ref[:, hh * DH:(hh + 1) * DH] = (
                    tot[:, hh * DH:(hh + 1) * DH]
                    / tot[:, DM + hh:DM + hh + 1])

            out_ref[0] = jnp.dot(ctx_ref[:, :], wo_ref[:, :],
                                 preferred_element_type=jnp.float32)

            r0.wait_send()
            l0.wait_send()
            r1.wait_send()
            l1.wait_send()

    kv_spec = pl.BlockSpec(
        (1, 16, 4, 64, DH), lambda h: (0, 0, 0, 0, h))
    return pl.pallas_call(
        body,
        grid=(HQ,),
        out_shape=jax.ShapeDtypeStruct((1, SQ, DM), jnp.float32),
        in_specs=[
            pl.BlockSpec((1, SQ, DM), lambda h: (0, 0, 0)),
            pl.BlockSpec((DM, DM), lambda h: (0, 0)),
            kv_spec,
            kv_spec,
            pl.BlockSpec((DM, DM), lambda h: (0, 0)),
        ],
        out_specs=pl.BlockSpec((1, SQ, DM), lambda h: (0, 0, 0)),
        scratch_shapes=[
            pltpu.VMEM((N_DEV, SQ, CW), jnp.bfloat16),
            pltpu.VMEM((SQ, DM), jnp.float32),
            pltpu.VMEM((SQ, DM), jnp.float32),
            pltpu.SemaphoreType.DMA((4,)),
            pltpu.SemaphoreType.DMA((4,)),
        ],
        compiler_params=pltpu.CompilerParams(
            collective_id=0,
            vmem_limit_bytes=100 * 1024 * 1024,
            dimension_semantics=("arbitrary",),
        ),
    )(x, Wq,
      K_ext.reshape(1, 16, 4, 64, HQ * DH),
      V_ext.reshape(1, 16, 4, 64, HQ * DH),
      Wo)
